# Initial kernel scaffold; baseline (speedup 1.0000x reference)
import jax
import jax.numpy as jnp
from jax.experimental import pallas as pl


def kernel(X, edge_index, W1, b1, W2, b2):
    raise NotImplementedError("write your pallas kernel here")


# speedup vs baseline: 3.6058x; 3.6058x over previous
import functools

import jax
import jax.numpy as jnp
from jax import lax
from jax.experimental import pallas as pl
from jax.experimental.pallas import tpu as pltpu
from jax.experimental.pallas import tpu_sc as plsc

N_NODES = 10000
N_PAD = 10240
N_EDGES = 320000

NUM_CORES = 2
NUM_SUBCORES = 16
NUM_WORKERS = NUM_CORES * NUM_SUBCORES

CHUNK = 80
CHUNKS_PER_WORKER = N_EDGES // CHUNK // NUM_WORKERS
ROWS_PER_TILE = N_PAD // NUM_SUBCORES
ZCOPIES = ROWS_PER_TILE // CHUNK


def _sc_agg_kernel(feat_hbm, src_hbm, dst_hbm, zrows_hbm, out_hbm,
                   acc_sh, src_v, dst_v, rows_v, sem, *, d):
    cid = lax.axis_index("c")
    sid = lax.axis_index("s")
    wid = cid * NUM_SUBCORES + sid

    r0 = sid * ROWS_PER_TILE
    pltpu.sync_copy(zrows_hbm, rows_v)
    for k in range(ZCOPIES):
        pltpu.sync_copy(rows_v, acc_sh.at[pl.ds(r0 + k * CHUNK, CHUNK), :])
    plsc.subcore_barrier()

    e0 = wid * CHUNKS_PER_WORKER * CHUNK

    def body(i, _):
        base = e0 + i * CHUNK
        pltpu.sync_copy(src_hbm.at[pl.ds(base, CHUNK)], src_v)
        pltpu.sync_copy(dst_hbm.at[pl.ds(base, CHUNK)], dst_v)
        pltpu.async_copy(feat_hbm.at[src_v], rows_v, sem).wait()
        pltpu.sync_copy(rows_v, acc_sh.at[dst_v], add=True)
        return 0

    lax.fori_loop(0, CHUNKS_PER_WORKER, body, 0)
    plsc.subcore_barrier()

    o0 = cid * N_PAD + r0
    for k in range(ZCOPIES):
        pltpu.sync_copy(acc_sh.at[pl.ds(r0 + k * CHUNK, CHUNK), :], rows_v)
        pltpu.sync_copy(rows_v, out_hbm.at[pl.ds(o0 + k * CHUNK, CHUNK), :])


def _make_sc_agg(d):
    mesh = plsc.VectorSubcoreMesh(core_axis_name="c", subcore_axis_name="s")
    out_type = jax.ShapeDtypeStruct((NUM_CORES * N_PAD, d), jnp.float32)
    scratch = [
        pltpu.VMEM_SHARED((N_PAD, d), jnp.float32),
        pltpu.VMEM((CHUNK,), jnp.int32),
        pltpu.VMEM((CHUNK,), jnp.int32),
        pltpu.VMEM((CHUNK, d), jnp.float32),
        pltpu.SemaphoreType.DMA,
    ]
    return pl.kernel(
        functools.partial(_sc_agg_kernel, d=d),
        out_type=out_type,
        mesh=mesh,
        scratch_types=scratch,
    )


def _tc1_kernel(x_ref, s1a_ref, s1b_ref, deg_ref, w1_ref, b1_ref, h_ref):
    rdeg = 1.0 / jnp.maximum(deg_ref[...], 1.0)
    a1 = (s1a_ref[...] + s1b_ref[...]) * rdeg
    x = x_ref[...]
    h = (jnp.dot(x, w1_ref[:128, :], preferred_element_type=jnp.float32)
         + jnp.dot(a1, w1_ref[128:, :], preferred_element_type=jnp.float32)
         + b1_ref[...])
    h_ref[...] = jnp.maximum(h, 0.0)


def _tc2_kernel(h_ref, s2a_ref, s2b_ref, deg_ref, w2_ref, b2_ref, out_ref):
    rdeg = 1.0 / jnp.maximum(deg_ref[...], 1.0)
    a2 = (s2a_ref[...] + s2b_ref[...]) * rdeg
    out_ref[...] = (
        jnp.dot(h_ref[...], w2_ref[:128, :], preferred_element_type=jnp.float32)
        + jnp.dot(a2, w2_ref[128:, :], preferred_element_type=jnp.float32)
        + b2_ref[...])


_TC_BLOCK = 1024


def _row_spec(d):
    return pl.BlockSpec((_TC_BLOCK, d), lambda i: (i, 0))


def _full_spec(shape):
    return pl.BlockSpec(shape, lambda i: tuple(0 for _ in shape))


def kernel(X, edge_index, W1, b1, W2, b2):
    src = edge_index[0]
    dst = edge_index[1]
    z128 = jnp.zeros((CHUNK, 128), jnp.float32)

    deg = jax.ops.segment_sum(jnp.ones((N_EDGES,), jnp.float32), dst,
                              num_segments=N_NODES)
    deg_col = deg.reshape(N_NODES, 1)

    s1 = _make_sc_agg(128)(X, src, dst, z128)
    s1 = s1.reshape(NUM_CORES, N_PAD, 128)

    grid = pl.cdiv(N_NODES, _TC_BLOCK)
    h = pl.pallas_call(
        _tc1_kernel,
        grid=(grid,),
        in_specs=[
            _row_spec(128), _row_spec(128), _row_spec(128), _row_spec(1),
            _full_spec((256, 128)), _full_spec((1, 128)),
        ],
        out_specs=_row_spec(128),
        out_shape=jax.ShapeDtypeStruct((N_NODES, 128), jnp.float32),
    )(X, s1[0], s1[1], deg_col, W1, b1.reshape(1, 128))

    s2 = _make_sc_agg(128)(h, src, dst, z128)
    s2 = s2.reshape(NUM_CORES, N_PAD, 128)

    out = pl.pallas_call(
        _tc2_kernel,
        grid=(grid,),
        in_specs=[
            _row_spec(128), _row_spec(128), _row_spec(128), _row_spec(1),
            _full_spec((256, 64)), _full_spec((1, 64)),
        ],
        out_specs=_row_spec(64),
        out_shape=jax.ShapeDtypeStruct((N_NODES, 64), jnp.float32),
    )(h, s2[0], s2[1], deg_col, W2, b2.reshape(1, 64))

    return out


# speedup vs baseline: 5.0221x; 1.3928x over previous
import functools

import jax
import jax.numpy as jnp
from jax import lax
from jax.experimental import pallas as pl
from jax.experimental.pallas import tpu as pltpu
from jax.experimental.pallas import tpu_sc as plsc

N_NODES = 10000
N_PAD = 10240
N_EDGES = 320000

NUM_CORES = 2
NUM_SUBCORES = 16
NUM_WORKERS = NUM_CORES * NUM_SUBCORES

CHUNK = 80
CHUNKS_PER_WORKER = N_EDGES // CHUNK // NUM_WORKERS
ROWS_PER_TILE = N_PAD // NUM_SUBCORES
ZCOPIES = ROWS_PER_TILE // CHUNK


def _sc_agg_kernel(feat_hbm, src_hbm, dst4_hbm, zrows_hbm, out_hbm,
                   acc_sh, src_all, dst3_v, rows_a, rows_b,
                   sga, sgb, ssa, ssb, *, d):
    cid = lax.axis_index("c")
    sid = lax.axis_index("s")
    wid = cid * NUM_SUBCORES + sid

    r0 = sid * ROWS_PER_TILE
    pltpu.sync_copy(zrows_hbm, rows_a)
    for k in range(ZCOPIES):
        pltpu.sync_copy(rows_a, acc_sh.at[pl.ds(r0 + k * CHUNK, CHUNK), :])

    e0 = wid * CHUNKS_PER_WORKER * CHUNK
    pltpu.sync_copy(src_hbm.at[pl.ds(e0, CHUNKS_PER_WORKER * CHUNK)],
                    src_all)
    pltpu.sync_copy(dst4_hbm.at[wid], dst3_v)
    plsc.subcore_barrier()

    npairs = pl.cdiv(CHUNKS_PER_WORKER, 2)

    def body(t, _):
        j0 = 2 * t
        j1 = 2 * t + 1
        ga = pltpu.async_copy(
            feat_hbm.at[src_all.at[pl.ds(j0 * CHUNK, CHUNK)]], rows_a, sga)

        @pl.when(j1 < CHUNKS_PER_WORKER)
        def _():
            gb = pltpu.async_copy(
                feat_hbm.at[src_all.at[pl.ds(j1 * CHUNK, CHUNK)]], rows_b,
                sgb)
            ga.wait()
            sa = pltpu.async_copy(rows_a, acc_sh.at[dst3_v.at[j0]], ssa,
                                  add=True)
            gb.wait()
            sb = pltpu.async_copy(rows_b, acc_sh.at[dst3_v.at[j1]], ssb,
                                  add=True)
            sa.wait()
            sb.wait()

        @pl.when(j1 >= CHUNKS_PER_WORKER)
        def _():
            ga.wait()
            pltpu.async_copy(rows_a, acc_sh.at[dst3_v.at[j0]], ssa,
                             add=True).wait()
        return 0

    lax.fori_loop(0, npairs, body, 0)
    plsc.subcore_barrier()

    o0 = cid * N_PAD + r0
    for k in range(ZCOPIES):
        pltpu.sync_copy(acc_sh.at[pl.ds(r0 + k * CHUNK, CHUNK), :], rows_a)
        pltpu.sync_copy(rows_a, out_hbm.at[pl.ds(o0 + k * CHUNK, CHUNK), :])


def _make_sc_agg(d):
    mesh = plsc.VectorSubcoreMesh(core_axis_name="c", subcore_axis_name="s")
    out_type = jax.ShapeDtypeStruct((NUM_CORES * N_PAD, d), jnp.float32)
    scratch = [
        pltpu.VMEM_SHARED((N_PAD, d), jnp.float32),
        pltpu.VMEM((CHUNKS_PER_WORKER * CHUNK,), jnp.int32),
        pltpu.VMEM((CHUNKS_PER_WORKER, CHUNK), jnp.int32),
        pltpu.VMEM((CHUNK, d), jnp.float32),
        pltpu.VMEM((CHUNK, d), jnp.float32),
        pltpu.SemaphoreType.DMA,
        pltpu.SemaphoreType.DMA,
        pltpu.SemaphoreType.DMA,
        pltpu.SemaphoreType.DMA,
    ]
    return pl.kernel(
        functools.partial(_sc_agg_kernel, d=d),
        out_type=out_type,
        mesh=mesh,
        scratch_types=scratch,
    )


def _tc1_kernel(x_ref, s1a_ref, s1b_ref, deg_ref, w1_ref, b1_ref, h_ref):
    rdeg = 1.0 / jnp.maximum(deg_ref[...], 1.0)
    a1 = (s1a_ref[...] + s1b_ref[...]) * rdeg
    x = x_ref[...]
    h = (jnp.dot(x, w1_ref[:128, :], preferred_element_type=jnp.float32)
         + jnp.dot(a1, w1_ref[128:, :], preferred_element_type=jnp.float32)
         + b1_ref[...])
    h_ref[...] = jnp.maximum(h, 0.0)


def _tc2_kernel(h_ref, s2a_ref, s2b_ref, deg_ref, w2_ref, b2_ref, out_ref):
    rdeg = 1.0 / jnp.maximum(deg_ref[...], 1.0)
    a2 = (s2a_ref[...] + s2b_ref[...]) * rdeg
    out_ref[...] = (
        jnp.dot(h_ref[...], w2_ref[:128, :], preferred_element_type=jnp.float32)
        + jnp.dot(a2, w2_ref[128:, :], preferred_element_type=jnp.float32)
        + b2_ref[...])


_TC_BLOCK = 1024


def _row_spec(d):
    return pl.BlockSpec((_TC_BLOCK, d), lambda i: (i, 0))


def _full_spec(shape):
    return pl.BlockSpec(shape, lambda i: tuple(0 for _ in shape))


def kernel(X, edge_index, W1, b1, W2, b2):
    src = edge_index[0]
    dst = edge_index[1]
    z128 = jnp.zeros((CHUNK, 128), jnp.float32)

    deg = jax.ops.segment_sum(jnp.ones((N_EDGES,), jnp.float32), dst,
                              num_segments=N_NODES)
    deg_col = deg.reshape(N_NODES, 1)

    dst4 = dst.reshape(NUM_WORKERS, CHUNKS_PER_WORKER, CHUNK)

    s1 = _make_sc_agg(128)(X, src, dst4, z128)
    s1 = s1.reshape(NUM_CORES, N_PAD, 128)

    grid = pl.cdiv(N_NODES, _TC_BLOCK)
    h = pl.pallas_call(
        _tc1_kernel,
        grid=(grid,),
        in_specs=[
            _row_spec(128), _row_spec(128), _row_spec(128), _row_spec(1),
            _full_spec((256, 128)), _full_spec((1, 128)),
        ],
        out_specs=_row_spec(128),
        out_shape=jax.ShapeDtypeStruct((N_NODES, 128), jnp.float32),
    )(X, s1[0], s1[1], deg_col, W1, b1.reshape(1, 128))

    s2 = _make_sc_agg(128)(h, src, dst4, z128)
    s2 = s2.reshape(NUM_CORES, N_PAD, 128)

    out = pl.pallas_call(
        _tc2_kernel,
        grid=(grid,),
        in_specs=[
            _row_spec(128), _row_spec(128), _row_spec(128), _row_spec(1),
            _full_spec((256, 64)), _full_spec((1, 64)),
        ],
        out_specs=_row_spec(64),
        out_shape=jax.ShapeDtypeStruct((N_NODES, 64), jnp.float32),
    )(h, s2[0], s2[1], deg_col, W2, b2.reshape(1, 64))

    return out


# speedup vs baseline: 8.0304x; 1.5990x over previous
import functools

import jax
import jax.numpy as jnp
from jax import lax
from jax.experimental import pallas as pl
from jax.experimental.pallas import tpu as pltpu
from jax.experimental.pallas import tpu_sc as plsc

N_NODES = 10000
N_PAD = 10240
N_EDGES = 320000

NUM_CORES = 2
NUM_SUBCORES = 16
NUM_WORKERS = NUM_CORES * NUM_SUBCORES

CHUNK = 80
CHUNKS_PER_WORKER = N_EDGES // CHUNK // NUM_WORKERS
ROWS_PER_TILE = N_PAD // NUM_SUBCORES
ZCOPIES = ROWS_PER_TILE // CHUNK


def _sc_agg_kernel(feat_hbm, src_hbm, dst4_hbm, zrows_hbm, ones_hbm,
                   out_hbm, deg_hbm,
                   acc_sh, src_all, dst3_v, rows_a, rows_b,
                   sga, sgb, ssa, ssb, *, d, with_deg):
    cid = lax.axis_index("c")
    sid = lax.axis_index("s")
    wid = cid * NUM_SUBCORES + sid

    r0 = sid * ROWS_PER_TILE
    pltpu.sync_copy(zrows_hbm, rows_a)
    for k in range(ZCOPIES):
        pltpu.sync_copy(rows_a, acc_sh.at[pl.ds(r0 + k * CHUNK, CHUNK), :])

    e0 = wid * CHUNKS_PER_WORKER * CHUNK
    pltpu.sync_copy(src_hbm.at[pl.ds(e0, CHUNKS_PER_WORKER * CHUNK)],
                    src_all)
    pltpu.sync_copy(dst4_hbm.at[wid], dst3_v)
    plsc.subcore_barrier()

    npairs = pl.cdiv(CHUNKS_PER_WORKER, 2)

    def body(t, _):
        j0 = 2 * t
        j1 = 2 * t + 1
        ga = pltpu.async_copy(
            feat_hbm.at[src_all.at[pl.ds(j0 * CHUNK, CHUNK)]], rows_a, sga)

        @pl.when(j1 < CHUNKS_PER_WORKER)
        def _():
            gb = pltpu.async_copy(
                feat_hbm.at[src_all.at[pl.ds(j1 * CHUNK, CHUNK)]], rows_b,
                sgb)
            ga.wait()
            sa = pltpu.async_copy(rows_a, acc_sh.at[dst3_v.at[j0]], ssa,
                                  add=True)
            gb.wait()
            sb = pltpu.async_copy(rows_b, acc_sh.at[dst3_v.at[j1]], ssb,
                                  add=True)
            sa.wait()
            sb.wait()

        @pl.when(j1 >= CHUNKS_PER_WORKER)
        def _():
            ga.wait()
            pltpu.async_copy(rows_a, acc_sh.at[dst3_v.at[j0]], ssa,
                             add=True).wait()
        return 0

    lax.fori_loop(0, npairs, body, 0)
    plsc.subcore_barrier()

    o0 = cid * N_PAD + r0
    for k in range(ZCOPIES):
        pltpu.sync_copy(acc_sh.at[pl.ds(r0 + k * CHUNK, CHUNK), :], rows_a)
        pltpu.sync_copy(rows_a, out_hbm.at[pl.ds(o0 + k * CHUNK, CHUNK), :])

    if with_deg:
        pltpu.sync_copy(zrows_hbm, rows_a)
        for k in range(ZCOPIES):
            pltpu.sync_copy(rows_a,
                            acc_sh.at[pl.ds(r0 + k * CHUNK, CHUNK), :])
        pltpu.sync_copy(ones_hbm, rows_b)
        plsc.subcore_barrier()

        def dbody(t, _):
            j0 = 2 * t
            j1 = 2 * t + 1
            sa = pltpu.async_copy(rows_b, acc_sh.at[dst3_v.at[j0]], ssa,
                                  add=True)

            @pl.when(j1 < CHUNKS_PER_WORKER)
            def _():
                sb = pltpu.async_copy(rows_b, acc_sh.at[dst3_v.at[j1]],
                                      ssb, add=True)
                sa.wait()
                sb.wait()

            @pl.when(j1 >= CHUNKS_PER_WORKER)
            def _():
                sa.wait()
            return 0

        lax.fori_loop(0, npairs, dbody, 0)
        plsc.subcore_barrier()

        for k in range(ZCOPIES):
            pltpu.sync_copy(acc_sh.at[pl.ds(r0 + k * CHUNK, CHUNK), :],
                            rows_a)
            pltpu.sync_copy(rows_a,
                            deg_hbm.at[pl.ds(o0 + k * CHUNK, CHUNK), :])


def _make_sc_agg(d, with_deg):
    mesh = plsc.VectorSubcoreMesh(core_axis_name="c", subcore_axis_name="s")
    out_type = [
        jax.ShapeDtypeStruct((NUM_CORES * N_PAD, d), jnp.float32),
        jax.ShapeDtypeStruct((NUM_CORES * N_PAD, d), jnp.float32),
    ]
    scratch = [
        pltpu.VMEM_SHARED((N_PAD, d), jnp.float32),
        pltpu.VMEM((CHUNKS_PER_WORKER * CHUNK,), jnp.int32),
        pltpu.VMEM((CHUNKS_PER_WORKER, CHUNK), jnp.int32),
        pltpu.VMEM((CHUNK, d), jnp.float32),
        pltpu.VMEM((CHUNK, d), jnp.float32),
        pltpu.SemaphoreType.DMA,
        pltpu.SemaphoreType.DMA,
        pltpu.SemaphoreType.DMA,
        pltpu.SemaphoreType.DMA,
    ]
    return pl.kernel(
        functools.partial(_sc_agg_kernel, d=d, with_deg=with_deg),
        out_type=out_type,
        mesh=mesh,
        scratch_types=scratch,
    )


def _tc1_kernel(x_ref, s1a_ref, s1b_ref, da_ref, db_ref, w1_ref, b1_ref,
                h_ref):
    rdeg = 1.0 / jnp.maximum(da_ref[...] + db_ref[...], 1.0)
    a1 = (s1a_ref[...] + s1b_ref[...]) * rdeg
    x = x_ref[...]
    h = (jnp.dot(x, w1_ref[:128, :], preferred_element_type=jnp.float32)
         + jnp.dot(a1, w1_ref[128:, :], preferred_element_type=jnp.float32)
         + b1_ref[...])
    h_ref[...] = jnp.maximum(h, 0.0)


def _tc2_kernel(h_ref, s2a_ref, s2b_ref, da_ref, db_ref, w2_ref, b2_ref,
                out_ref):
    rdeg = 1.0 / jnp.maximum(da_ref[...] + db_ref[...], 1.0)
    a2 = (s2a_ref[...] + s2b_ref[...]) * rdeg
    out_ref[...] = (
        jnp.dot(h_ref[...], w2_ref[:128, :], preferred_element_type=jnp.float32)
        + jnp.dot(a2, w2_ref[128:, :], preferred_element_type=jnp.float32)
        + b2_ref[...])


_TC_BLOCK = 1024


def _row_spec(d):
    return pl.BlockSpec((_TC_BLOCK, d), lambda i: (i, 0))


def _full_spec(shape):
    return pl.BlockSpec(shape, lambda i: tuple(0 for _ in shape))


def kernel(X, edge_index, W1, b1, W2, b2):
    src = edge_index[0]
    dst = edge_index[1]
    z128 = jnp.zeros((CHUNK, 128), jnp.float32)

    ones128 = jnp.ones((CHUNK, 128), jnp.float32)
    dst4 = dst.reshape(NUM_WORKERS, CHUNKS_PER_WORKER, CHUNK)

    s1, degp = _make_sc_agg(128, True)(X, src, dst4, z128, ones128)
    s1 = s1.reshape(NUM_CORES, N_PAD, 128)
    degp = degp.reshape(NUM_CORES, N_PAD, 128)
    da = degp[0, :N_NODES, 0:1]
    db = degp[1, :N_NODES, 0:1]

    grid = pl.cdiv(N_NODES, _TC_BLOCK)
    h = pl.pallas_call(
        _tc1_kernel,
        grid=(grid,),
        in_specs=[
            _row_spec(128), _row_spec(128), _row_spec(128), _row_spec(1),
            _row_spec(1),
            _full_spec((256, 128)), _full_spec((1, 128)),
        ],
        out_specs=_row_spec(128),
        out_shape=jax.ShapeDtypeStruct((N_NODES, 128), jnp.float32),
    )(X, s1[0], s1[1], da, db, W1, b1.reshape(1, 128))

    s2, _ = _make_sc_agg(128, False)(h, src, dst4, z128, ones128)
    s2 = s2.reshape(NUM_CORES, N_PAD, 128)

    out = pl.pallas_call(
        _tc2_kernel,
        grid=(grid,),
        in_specs=[
            _row_spec(128), _row_spec(128), _row_spec(128), _row_spec(1),
            _row_spec(1),
            _full_spec((256, 64)), _full_spec((1, 64)),
        ],
        out_specs=_row_spec(64),
        out_shape=jax.ShapeDtypeStruct((N_NODES, 64), jnp.float32),
    )(h, s2[0], s2[1], da, db, W2, b2.reshape(1, 64))

    return out


# speedup vs baseline: 8.0614x; 1.0039x over previous
import functools

import jax
import jax.numpy as jnp
from jax import lax
from jax.experimental import pallas as pl
from jax.experimental.pallas import tpu as pltpu
from jax.experimental.pallas import tpu_sc as plsc

N_NODES = 10000
N_PAD = 10240
N_EDGES = 320000

NUM_CORES = 2
NUM_SUBCORES = 16
NUM_WORKERS = NUM_CORES * NUM_SUBCORES

CHUNK = 80
CHUNKS_PER_WORKER = N_EDGES // CHUNK // NUM_WORKERS
ROWS_PER_TILE = N_PAD // NUM_SUBCORES
ZCOPIES = ROWS_PER_TILE // CHUNK


NBUF = 2
NHALF = 2
HC = CHUNK // NHALF


def _sc_agg_kernel(feat_hbm, src_hbm, dst4_hbm, zrows_hbm, ones_hbm,
                   out_hbm, deg_hbm,
                   acc_sh, src_all, dst3_v, rows, sg, ss,
                   *, d, with_deg):
    cid = lax.axis_index("c")
    sid = lax.axis_index("s")
    wid = cid * NUM_SUBCORES + sid

    r0 = sid * ROWS_PER_TILE
    pltpu.sync_copy(zrows_hbm, rows[0])
    for k in range(ZCOPIES):
        pltpu.sync_copy(rows[0],
                        acc_sh.at[pl.ds(r0 + k * CHUNK, CHUNK), :])
    del r0

    e0 = wid * CHUNKS_PER_WORKER * CHUNK
    pltpu.sync_copy(src_hbm.at[pl.ds(e0, CHUNKS_PER_WORKER * CHUNK)],
                    src_all)
    pltpu.sync_copy(dst4_hbm.at[wid], dst3_v)
    plsc.subcore_barrier()

    nsteps = CHUNKS_PER_WORKER // NBUF

    def body(t, _):
        j0 = NBUF * t
        gs = []
        for b in range(NBUF):
            for hh in range(NHALF):
                gs.append(pltpu.async_copy(
                    feat_hbm.at[src_all.at[
                        pl.ds((j0 + b) * CHUNK + hh * HC, HC)]],
                    rows[b].at[pl.ds(hh * HC, HC), :], sg[b * NHALF + hh]))
        scs = []
        for b in range(NBUF):
            for hh in range(NHALF):
                gs[b * NHALF + hh].wait()
            scs.append(pltpu.async_copy(
                rows[b], acc_sh.at[dst3_v.at[j0 + b]], ss[b], add=True))
        for s in scs:
            s.wait()
        return 0

    lax.fori_loop(0, nsteps, body, 0)
    plsc.subcore_barrier()
    r0 = sid * ROWS_PER_TILE

    o0 = cid * N_PAD + r0
    for k in range(ZCOPIES):
        pltpu.sync_copy(acc_sh.at[pl.ds(r0 + k * CHUNK, CHUNK), :],
                        rows[k % NBUF])
        pltpu.sync_copy(rows[k % NBUF],
                        out_hbm.at[pl.ds(o0 + k * CHUNK, CHUNK), :])

    if with_deg:
        pltpu.sync_copy(zrows_hbm, rows[0])
        for k in range(ZCOPIES):
            pltpu.sync_copy(rows[0],
                            acc_sh.at[pl.ds(r0 + k * CHUNK, CHUNK), :])
        pltpu.sync_copy(ones_hbm, rows[1])
        plsc.subcore_barrier()

        def dbody(t, _):
            j0 = NBUF * t
            scs = [pltpu.async_copy(
                       rows[1], acc_sh.at[dst3_v.at[j0 + b]], ss[b],
                       add=True)
                   for b in range(NBUF)]
            for s in scs:
                s.wait()
            return 0

        lax.fori_loop(0, nsteps, dbody, 0)
        plsc.subcore_barrier()

        for k in range(ZCOPIES):
            pltpu.sync_copy(acc_sh.at[pl.ds(r0 + k * CHUNK, CHUNK), :],
                            rows[k % NBUF])
            pltpu.sync_copy(rows[k % NBUF],
                            deg_hbm.at[pl.ds(o0 + k * CHUNK, CHUNK), :])


def _make_sc_agg(d, with_deg):
    mesh = plsc.VectorSubcoreMesh(core_axis_name="c", subcore_axis_name="s")
    out_type = [
        jax.ShapeDtypeStruct((NUM_CORES * N_PAD, d), jnp.float32),
        jax.ShapeDtypeStruct((NUM_CORES * N_PAD, d), jnp.float32),
    ]
    scratch = [
        pltpu.VMEM_SHARED((N_PAD, d), jnp.float32),
        pltpu.VMEM((CHUNKS_PER_WORKER * CHUNK,), jnp.int32),
        pltpu.VMEM((CHUNKS_PER_WORKER, CHUNK), jnp.int32),
        [pltpu.VMEM((CHUNK, d), jnp.float32)] * NBUF,
        [pltpu.SemaphoreType.DMA] * (NBUF * NHALF),
        [pltpu.SemaphoreType.DMA] * NBUF,
    ]
    return pl.kernel(
        functools.partial(_sc_agg_kernel, d=d, with_deg=with_deg),
        out_type=out_type,
        mesh=mesh,
        scratch_types=scratch,
    )


def _tc1_kernel(x_ref, s1a_ref, s1b_ref, da_ref, db_ref, w1_ref, b1_ref,
                h_ref):
    rdeg = 1.0 / jnp.maximum(da_ref[...] + db_ref[...], 1.0)
    a1 = (s1a_ref[...] + s1b_ref[...]) * rdeg
    x = x_ref[...]
    h = (jnp.dot(x, w1_ref[:128, :], preferred_element_type=jnp.float32)
         + jnp.dot(a1, w1_ref[128:, :], preferred_element_type=jnp.float32)
         + b1_ref[...])
    h_ref[...] = jnp.maximum(h, 0.0)


def _tc2_kernel(h_ref, s2a_ref, s2b_ref, da_ref, db_ref, w2_ref, b2_ref,
                out_ref):
    rdeg = 1.0 / jnp.maximum(da_ref[...] + db_ref[...], 1.0)
    a2 = (s2a_ref[...] + s2b_ref[...]) * rdeg
    out_ref[...] = (
        jnp.dot(h_ref[...], w2_ref[:128, :], preferred_element_type=jnp.float32)
        + jnp.dot(a2, w2_ref[128:, :], preferred_element_type=jnp.float32)
        + b2_ref[...])


_TC_BLOCK = 1024


def _row_spec(d):
    return pl.BlockSpec((_TC_BLOCK, d), lambda i: (i, 0))


def _full_spec(shape):
    return pl.BlockSpec(shape, lambda i: tuple(0 for _ in shape))


def kernel(X, edge_index, W1, b1, W2, b2):
    src = edge_index[0]
    dst = edge_index[1]
    z128 = jnp.zeros((CHUNK, 128), jnp.float32)

    ones128 = jnp.ones((CHUNK, 128), jnp.float32)
    dst4 = dst.reshape(NUM_WORKERS, CHUNKS_PER_WORKER, CHUNK)

    s1, degp = _make_sc_agg(128, True)(X, src, dst4, z128, ones128)
    s1 = s1.reshape(NUM_CORES, N_PAD, 128)
    degp = degp.reshape(NUM_CORES, N_PAD, 128)
    da = degp[0, :N_NODES, 0:1]
    db = degp[1, :N_NODES, 0:1]

    grid = pl.cdiv(N_NODES, _TC_BLOCK)
    h = pl.pallas_call(
        _tc1_kernel,
        grid=(grid,),
        in_specs=[
            _row_spec(128), _row_spec(128), _row_spec(128), _row_spec(1),
            _row_spec(1),
            _full_spec((256, 128)), _full_spec((1, 128)),
        ],
        out_specs=_row_spec(128),
        out_shape=jax.ShapeDtypeStruct((N_NODES, 128), jnp.float32),
    )(X, s1[0], s1[1], da, db, W1, b1.reshape(1, 128))

    s2, _ = _make_sc_agg(128, False)(h, src, dst4, z128, ones128)
    s2 = s2.reshape(NUM_CORES, N_PAD, 128)

    out = pl.pallas_call(
        _tc2_kernel,
        grid=(grid,),
        in_specs=[
            _row_spec(128), _row_spec(128), _row_spec(128), _row_spec(1),
            _row_spec(1),
            _full_spec((256, 64)), _full_spec((1, 64)),
        ],
        out_specs=_row_spec(64),
        out_shape=jax.ShapeDtypeStruct((N_NODES, 64), jnp.float32),
    )(h, s2[0], s2[1], da, db, W2, b2.reshape(1, 64))

    return out
